# baseline (speedup 1.0000x reference)
import functools

import jax
import jax.numpy as jnp
import numpy as np
from jax import lax
from jax.experimental import pallas as pl
from jax.experimental.pallas import tpu as pltpu
from jax.experimental.pallas import tpu_sc as plsc

B, N, D, K = 32, 576, 768, 432
U = N - K
CHUNK = 72
NMC = K // CHUNK
NUC = U // CHUNK


@functools.lru_cache(maxsize=1)
def _constants():
    with jax.ensure_compile_time_eval():
        u = jax.random.uniform(jax.random.key(42), (B, N))
        idx = np.asarray(jax.device_get(jnp.argsort(u, axis=-1)))
    base = np.arange(B, dtype=np.int64)[:, None] * N
    midx = (base + idx[:, :K]).reshape(B, NMC, CHUNK).astype(np.int32)
    uidx = (base + np.sort(idx[:, K:], axis=-1)
            ).reshape(B, NUC, CHUNK).astype(np.int32)
    lidx = idx[:, :K].astype(np.int32).reshape(B, 1, K)
    return midx, uidx, lidx


def _images_kernel(flat_patches, tokens, midx, uidx):
    info = plsc.get_sparse_core_info()
    nc = info.num_cores

    @functools.partial(
        pl.kernel,
        mesh=plsc.VectorSubcoreMesh(core_axis_name="c", subcore_axis_name="s"),
        out_type=jax.ShapeDtypeStruct((B * N, D), jnp.float32),
        scratch_types=[
            pltpu.VMEM((NMC, CHUNK), jnp.int32),
            pltpu.VMEM((NUC, CHUNK), jnp.int32),
            pltpu.VMEM((CHUNK, D), jnp.float32),
            pltpu.VMEM((CHUNK, D), jnp.float32),
            pltpu.SemaphoreType.DMA,
            pltpu.SemaphoreType.DMA,
            pltpu.SemaphoreType.DMA,
            pltpu.SemaphoreType.DMA,
        ],
    )
    def k(patches_hbm, tokens_hbm, midx_hbm, uidx_hbm, images_hbm,
          midx_v, uidx_v, tok_v, buf, gsem, wsem, tsem, fsem):
        wid = lax.axis_index("s") * nc + lax.axis_index("c")
        pltpu.sync_copy(midx_hbm.at[wid], midx_v)
        pltpu.sync_copy(uidx_hbm.at[wid], uidx_v)
        fill = pltpu.async_copy(tokens_hbm, tok_v, fsem)
        g = pltpu.async_copy(patches_hbm.at[uidx_v.at[0]], buf, gsem)
        fill.wait()
        tsc = [pltpu.async_copy(tok_v, images_hbm.at[midx_v.at[j]], tsem)
               for j in range(NMC)]
        for j in range(NUC):
            g.wait()
            pltpu.async_copy(buf, images_hbm.at[uidx_v.at[j]], wsem).wait()
            if j + 1 < NUC:
                g = pltpu.async_copy(
                    patches_hbm.at[uidx_v.at[j + 1]], buf, gsem)
        for c in tsc:
            c.wait()

    return k(flat_patches, tokens, midx, uidx)


def _mp_body(lidx_ref, patches_ref, out_ref):
    sel = lidx_ref[0, 0, :]
    onehot_t = (lax.broadcasted_iota(jnp.int32, (N, K), 0)
                == sel[None, :]).astype(jnp.float32)
    out_ref[0] = lax.dot_general(
        onehot_t, patches_ref[0],
        dimension_numbers=(((0,), (0,)), ((), ())),
        preferred_element_type=jnp.float32)


def _mp_kernel(patches, lidx):
    return pl.pallas_call(
        _mp_body,
        grid=(B,),
        in_specs=[
            pl.BlockSpec((1, 1, K), lambda b: (b, 0, 0)),
            pl.BlockSpec((1, N, D), lambda b: (b, 0, 0)),
        ],
        out_specs=pl.BlockSpec((1, K, D), lambda b: (b, 0, 0)),
        out_shape=jax.ShapeDtypeStruct((B, K, D), jnp.float32),
    )(lidx, patches)


def kernel(patches, mask):
    midx_np, uidx_np, lidx_np = _constants()
    midx = jnp.asarray(midx_np)
    uidx = jnp.asarray(uidx_np)
    lidx = jnp.asarray(lidx_np)
    tokens = jnp.broadcast_to(mask, (CHUNK, D))
    flat = patches.reshape(B * N, D)
    images = _images_kernel(flat, tokens, midx, uidx)
    masked_patches = _mp_kernel(patches, lidx)
    return (images.reshape(B, N, D), masked_patches)


# speedup vs baseline: 1.0340x; 1.0340x over previous
import functools

import jax
import jax.numpy as jnp
import numpy as np
from jax import lax
from jax.experimental import pallas as pl
from jax.experimental.pallas import tpu as pltpu
from jax.experimental.pallas import tpu_sc as plsc

B, N, D, K = 32, 576, 768, 432
U = N - K
CHUNK = 48
NMC = K // CHUNK
NUC = U // CHUNK


@functools.lru_cache(maxsize=1)
def _constants():
    with jax.ensure_compile_time_eval():
        u = jax.random.uniform(jax.random.key(42), (B, N))
        idx = np.asarray(jax.device_get(jnp.argsort(u, axis=-1)))
    base = np.arange(B, dtype=np.int64)[:, None] * N
    midx = (base + idx[:, :K]).reshape(B, NMC, CHUNK).astype(np.int32)
    uidx = (base + np.sort(idx[:, K:], axis=-1)
            ).reshape(B, NUC, CHUNK).astype(np.int32)
    return midx, uidx


def _sc_kernel(flat_patches, tokens, midx, uidx):
    info = plsc.get_sparse_core_info()
    nc = info.num_cores

    @functools.partial(
        pl.kernel,
        mesh=plsc.VectorSubcoreMesh(core_axis_name="c", subcore_axis_name="s"),
        out_type=(
            jax.ShapeDtypeStruct((B * N, D), jnp.float32),
            jax.ShapeDtypeStruct((B * K, D), jnp.float32),
        ),
        scratch_types=[
            pltpu.VMEM((NMC, CHUNK), jnp.int32),
            pltpu.VMEM((NUC, CHUNK), jnp.int32),
            pltpu.VMEM((CHUNK, D), jnp.float32),
            pltpu.VMEM((2, CHUNK, D), jnp.float32),
            pltpu.SemaphoreType.DMA,
            pltpu.SemaphoreType.DMA,
            pltpu.SemaphoreType.DMA,
            pltpu.SemaphoreType.DMA,
            pltpu.SemaphoreType.DMA,
        ],
    )
    def k(patches_hbm, tokens_hbm, midx_hbm, uidx_hbm, images_hbm, mp_hbm,
          midx_v, uidx_v, tok_v, bufs, g0, g1, w0, w1, tsem):
        wid = lax.axis_index("s") * nc + lax.axis_index("c")
        gsems, wsems = (g0, g1), (w0, w1)
        pltpu.sync_copy(midx_hbm.at[wid], midx_v)
        pltpu.sync_copy(uidx_hbm.at[wid], uidx_v)
        fill = pltpu.async_copy(tokens_hbm, tok_v, tsem)
        g = pltpu.async_copy(patches_hbm.at[midx_v.at[0]], bufs.at[0],
                             gsems[0])
        fill.wait()
        tsc = [pltpu.async_copy(tok_v, images_hbm.at[midx_v.at[j]], tsem)
               for j in range(NMC)]
        w = [None] * NMC
        for j in range(NMC):
            b = j % 2
            g.wait()
            if j + 1 < NMC:
                if j >= 1:
                    w[j - 1].wait()
                g = pltpu.async_copy(
                    patches_hbm.at[midx_v.at[j + 1]], bufs.at[1 - b],
                    gsems[1 - b])
            w[j] = pltpu.async_copy(
                bufs.at[b], mp_hbm.at[pl.ds(wid * K + j * CHUNK, CHUNK)],
                wsems[b])
        w[NMC - 2].wait()
        w[NMC - 1].wait()
        s = [None] * NUC
        g = pltpu.async_copy(patches_hbm.at[uidx_v.at[0]], bufs.at[0],
                             gsems[0])
        for j in range(NUC):
            b = j % 2
            g.wait()
            if j + 1 < NUC:
                if j >= 1:
                    s[j - 1].wait()
                g = pltpu.async_copy(
                    patches_hbm.at[uidx_v.at[j + 1]], bufs.at[1 - b],
                    gsems[1 - b])
            s[j] = pltpu.async_copy(
                bufs.at[b], images_hbm.at[uidx_v.at[j]], wsems[b])
        s[NUC - 2].wait()
        s[NUC - 1].wait()
        for c in tsc:
            c.wait()

    return k(flat_patches, tokens, midx, uidx)


def kernel(patches, mask):
    midx_np, uidx_np = _constants()
    midx = jnp.asarray(midx_np)
    uidx = jnp.asarray(uidx_np)
    tokens = jnp.broadcast_to(mask, (CHUNK, D))
    flat = patches.reshape(B * N, D)
    images, mp = _sc_kernel(flat, tokens, midx, uidx)
    return (images.reshape(B, N, D), mp.reshape(B, K, D))


# speedup vs baseline: 1.0890x; 1.0532x over previous
import functools

import jax
import jax.numpy as jnp
import numpy as np
from jax import lax
from jax.experimental import pallas as pl
from jax.experimental.pallas import tpu as pltpu
from jax.experimental.pallas import tpu_sc as plsc

B, N, D, K = 32, 576, 768, 432
NCHUNK = 6
CHUNK = K // NCHUNK
MB = 4


@functools.lru_cache(maxsize=1)
def _constants():
    with jax.ensure_compile_time_eval():
        u = jax.random.uniform(jax.random.key(42), (B, N))
        idx = np.asarray(jax.device_get(jnp.argsort(u, axis=-1)))
    mask_idx = idx[:, :K].astype(np.int32)
    flags = np.zeros((B, N), np.int32)
    flags[np.arange(B)[:, None], mask_idx] = 1
    gidx = (np.arange(B, dtype=np.int32)[:, None] * N + mask_idx)
    gidx = gidx.reshape(B, NCHUNK, CHUNK).astype(np.int32)
    return flags.reshape(B // MB, 1, MB * N), gidx


def _select_body(flags_ref, mask_ref, patches_ref, out_ref):
    flag = flags_ref[0, 0, :]
    out_ref[...] = jnp.where(flag[:, None] != 0,
                             mask_ref[0][None, :], patches_ref[...])


def _masked_images(patches, mask, flags):
    p2 = patches.reshape(B // MB, MB * N, D)
    out = pl.pallas_call(
        _select_body,
        grid=(B // MB,),
        in_specs=[
            pl.BlockSpec((1, 1, MB * N), lambda b: (b, 0, 0)),
            pl.BlockSpec((1, D), lambda b: (0, 0)),
            pl.BlockSpec((1, MB * N, D), lambda b: (b, 0, 0)),
        ],
        out_specs=pl.BlockSpec((1, MB * N, D), lambda b: (b, 0, 0)),
        out_shape=jax.ShapeDtypeStruct((B // MB, MB * N, D), jnp.float32),
    )(flags, mask, p2)
    return out.reshape(B, N, D)


def _gather_kernel(flat_patches, gidx):
    info = plsc.get_sparse_core_info()
    nc = info.num_cores

    @functools.partial(
        pl.kernel,
        mesh=plsc.VectorSubcoreMesh(core_axis_name="c", subcore_axis_name="s"),
        out_type=jax.ShapeDtypeStruct((B * K, D), jnp.float32),
        scratch_types=[
            pltpu.VMEM((NCHUNK, CHUNK), jnp.int32),
            pltpu.VMEM((2, CHUNK, D), jnp.float32),
            pltpu.SemaphoreType.DMA,
            pltpu.SemaphoreType.DMA,
            pltpu.SemaphoreType.DMA,
            pltpu.SemaphoreType.DMA,
        ],
    )
    def k(patches_hbm, gidx_hbm, out_hbm, idx_v, bufs, g0, g1, s0, s1):
        wid = lax.axis_index("s") * nc + lax.axis_index("c")
        pltpu.sync_copy(gidx_hbm.at[wid], idx_v)
        gsems, ssems = (g0, g1), (s0, s1)
        g = [None] * NCHUNK
        s = [None] * NCHUNK
        g[0] = pltpu.async_copy(patches_hbm.at[idx_v.at[0]], bufs.at[0],
                                gsems[0])
        for j in range(NCHUNK):
            b = j % 2
            g[j].wait()
            if j + 1 < NCHUNK:
                if j >= 1:
                    s[j - 1].wait()
                g[j + 1] = pltpu.async_copy(
                    patches_hbm.at[idx_v.at[j + 1]], bufs.at[1 - b],
                    gsems[1 - b])
            s[j] = pltpu.async_copy(
                bufs.at[b], out_hbm.at[pl.ds(wid * K + j * CHUNK, CHUNK)],
                ssems[b])
        s[NCHUNK - 2].wait()
        s[NCHUNK - 1].wait()

    return k(flat_patches, gidx)


def kernel(patches, mask):
    flags_np, gidx_np = _constants()
    flags = jnp.asarray(flags_np)
    gidx = jnp.asarray(gidx_np)
    flat = patches.reshape(B * N, D)
    masked_patches = _gather_kernel(flat, gidx).reshape(B, K, D)
    masked_images = _masked_images(patches, mask, flags)
    return (masked_images, masked_patches)
